# Initial kernel scaffold; baseline (speedup 1.0000x reference)
import jax
import jax.numpy as jnp
from jax.experimental import pallas as pl


def kernel(locF, Ck):
    raise NotImplementedError("write your pallas kernel here")


# speedup vs baseline: 4.0679x; 4.0679x over previous
import functools

import jax
import jax.numpy as jnp
from jax import lax
from jax.experimental import pallas as pl
from jax.experimental.pallas import tpu as pltpu
from jax.experimental.pallas import tpu_sc as plsc

N = 32768
K = 512
D = 32

NC = 2
NS = 16
NW = NC * NS
RPW = N // NW
KD = K * D

LANE_BLK = 4096
NBLK = N // LANE_BLK


def _assign_body(c_ref, xt_ref, out_ref):
    c = c_ref[...]
    xt = xt_ref[...]
    g = jnp.dot(c, xt, preferred_element_type=jnp.float32,
                precision=lax.Precision.HIGHEST)
    cn = jnp.sum(c * c, axis=1)
    score = cn[:, None] - 2.0 * g
    m = jnp.min(score, axis=0, keepdims=True)
    ids = lax.broadcasted_iota(jnp.int32, score.shape, 0)
    idx = jnp.min(jnp.where(score == m, ids, jnp.int32(K)), axis=0)
    out_ref[...] = idx.reshape(1, LANE_BLK)


_assign = pl.pallas_call(
    _assign_body,
    grid=(NBLK,),
    in_specs=[
        pl.BlockSpec((K, D), lambda i: (0, 0)),
        pl.BlockSpec((D, LANE_BLK), lambda i: (0, i)),
    ],
    out_specs=pl.BlockSpec((1, LANE_BLK), lambda i: (0, i)),
    out_shape=jax.ShapeDtypeStruct((1, N), jnp.int32),
)


@functools.cache
def _get_segsum():
    mesh = plsc.VectorSubcoreMesh(
        core_axis_name="c", subcore_axis_name="s", num_cores=NC, num_subcores=NS
    )

    @functools.partial(
        pl.kernel,
        mesh=mesh,
        out_type=[
            jax.ShapeDtypeStruct((NW, KD), jnp.float32),
            jax.ShapeDtypeStruct((NW, K), jnp.float32),
        ],
        scratch_types=[
            pltpu.VMEM((RPW * D,), jnp.float32),
            pltpu.VMEM((RPW,), jnp.int32),
            pltpu.VMEM((KD,), jnp.float32),
            pltpu.VMEM((K,), jnp.float32),
        ],
        compiler_params=pltpu.CompilerParams(needs_layout_passes=False),
    )
    def _segsum(x_hbm, idx_hbm, sums_hbm, cnt_hbm, x_v, idx_v, acc_v, cnt_v):
        wid = lax.axis_index("s") * NC + lax.axis_index("c")
        base = wid * RPW
        pltpu.sync_copy(x_hbm.at[pl.ds(base * D, RPW * D)], x_v)
        pltpu.sync_copy(idx_hbm.at[pl.ds(base, RPW)], idx_v)

        zeros16 = jnp.zeros((16,), jnp.float32)

        @plsc.parallel_loop(0, KD // 16, unroll=8)
        def _zero_acc(i):
            acc_v[pl.ds(i * 16, 16)] = zeros16

        @plsc.parallel_loop(0, K // 16, unroll=8)
        def _zero_cnt(i):
            cnt_v[pl.ds(i * 16, 16)] = zeros16

        lane = lax.iota(jnp.int32, 16)
        ones16 = jnp.ones((16,), jnp.float32)

        @plsc.parallel_loop(0, RPW // 16, unroll=2)
        def _group(g):
            idx16 = idx_v[pl.ds(g * 16, 16)]
            row_base = (g * 16 + lane) * D
            seg_base = idx16 * D
            for col in range(D):
                vals = plsc.load_gather(x_v, [row_base + col])
                plsc.addupdate_scatter(acc_v, [seg_base + col], vals)
            plsc.addupdate_scatter(cnt_v, [idx16], ones16)

        pltpu.sync_copy(acc_v, sums_hbm.at[wid])
        pltpu.sync_copy(cnt_v, cnt_hbm.at[wid])

    return _segsum


def _combine_body(sums_ref, cnt_ref, ck1_ref, n_ref):
    ck1_ref[...] = jnp.sum(sums_ref[...], axis=0, keepdims=True)
    n_ref[...] = jnp.sum(cnt_ref[...], axis=0, keepdims=True).astype(jnp.int32)


_combine = pl.pallas_call(
    _combine_body,
    out_shape=[
        jax.ShapeDtypeStruct((1, KD), jnp.float32),
        jax.ShapeDtypeStruct((1, K), jnp.int32),
    ],
)


def kernel(locF, Ck):
    X = locF[:, 0, :]
    idx = _assign(Ck[0], X.T).reshape(N)
    sums_p, cnt_p = _get_segsum()(X.reshape(N * D), idx)
    ck1, n_items = _combine(sums_p, cnt_p)
    return (ck1.reshape(1, K, D), n_items)


# speedup vs baseline: 4.5400x; 1.1161x over previous
import functools

import jax
import jax.numpy as jnp
from jax import lax
from jax.experimental import pallas as pl
from jax.experimental.pallas import tpu as pltpu
from jax.experimental.pallas import tpu_sc as plsc

N = 32768
K = 512
D = 32

NC = 2
NS = 16
NW = NC * NS
RPW = N // NW
KD = K * D

LANE_BLK = 4096
NH = N // 2
NBLK_H = NH // LANE_BLK
RPW_H = NH // NW


def _assign_body(c_ref, xt_ref, out_ref):
    c = c_ref[...]
    xt = xt_ref[...]
    g = jnp.dot(c, xt, preferred_element_type=jnp.float32,
                precision=lax.Precision.HIGHEST)
    cn = jnp.sum(c * c, axis=1)
    score = cn[:, None] - 2.0 * g
    m = jnp.min(score, axis=0, keepdims=True)
    ids = lax.broadcasted_iota(jnp.int32, score.shape, 0)
    idx = jnp.min(jnp.where(score == m, ids, jnp.int32(K)), axis=0)
    out_ref[...] = idx.reshape(1, LANE_BLK)


def _make_assign(blk0):
    return pl.pallas_call(
        _assign_body,
        grid=(NBLK_H,),
        in_specs=[
            pl.BlockSpec((K, D), lambda i: (0, 0)),
            pl.BlockSpec((D, LANE_BLK), lambda i: (0, blk0 + i)),
        ],
        out_specs=pl.BlockSpec((1, LANE_BLK), lambda i: (0, i)),
        out_shape=jax.ShapeDtypeStruct((1, NH), jnp.int32),
    )


_assign_lo = _make_assign(0)
_assign_hi = _make_assign(NBLK_H)


@functools.cache
def _get_segsum(row0):
    mesh = plsc.VectorSubcoreMesh(
        core_axis_name="c", subcore_axis_name="s", num_cores=NC, num_subcores=NS
    )

    @functools.partial(
        pl.kernel,
        mesh=mesh,
        out_type=[
            jax.ShapeDtypeStruct((NW, KD), jnp.float32),
            jax.ShapeDtypeStruct((NW, K), jnp.float32),
        ],
        scratch_types=[
            pltpu.VMEM((RPW_H * D,), jnp.float32),
            pltpu.VMEM((RPW_H,), jnp.int32),
            pltpu.VMEM((KD,), jnp.float32),
            pltpu.VMEM((K,), jnp.float32),
        ],
        compiler_params=pltpu.CompilerParams(needs_layout_passes=False),
    )
    def _segsum(x_hbm, idx_hbm, sums_hbm, cnt_hbm, x_v, idx_v, acc_v, cnt_v):
        wid = lax.axis_index("s") * NC + lax.axis_index("c")
        base = wid * RPW_H
        pltpu.sync_copy(x_hbm.at[pl.ds((row0 + base) * D, RPW_H * D)], x_v)
        pltpu.sync_copy(idx_hbm.at[pl.ds(base, RPW_H)], idx_v)

        zeros16 = jnp.zeros((16,), jnp.float32)

        @plsc.parallel_loop(0, KD // 16, unroll=8)
        def _zero_acc(i):
            acc_v[pl.ds(i * 16, 16)] = zeros16

        @plsc.parallel_loop(0, K // 16, unroll=8)
        def _zero_cnt(i):
            cnt_v[pl.ds(i * 16, 16)] = zeros16

        lane = lax.iota(jnp.int32, 16)
        ones16 = jnp.ones((16,), jnp.float32)

        @plsc.parallel_loop(0, RPW_H // 16, unroll=2)
        def _group(g):
            idx16 = idx_v[pl.ds(g * 16, 16)]
            row_base = (g * 16 + lane) * D
            seg_base = idx16 * D
            for col in range(D):
                vals = plsc.load_gather(x_v, [row_base + col])
                plsc.addupdate_scatter(acc_v, [seg_base + col], vals)
            plsc.addupdate_scatter(cnt_v, [idx16], ones16)

        pltpu.sync_copy(acc_v, sums_hbm.at[wid])
        pltpu.sync_copy(cnt_v, cnt_hbm.at[wid])

    return _segsum


def _combine_body(s1_ref, s2_ref, c1_ref, c2_ref, ck1_ref, n_ref):
    ck1_ref[...] = (jnp.sum(s1_ref[...], axis=0, keepdims=True)
                    + jnp.sum(s2_ref[...], axis=0, keepdims=True))
    n_ref[...] = (jnp.sum(c1_ref[...], axis=0, keepdims=True)
                  + jnp.sum(c2_ref[...], axis=0, keepdims=True)
                  ).astype(jnp.int32)


_combine = pl.pallas_call(
    _combine_body,
    out_shape=[
        jax.ShapeDtypeStruct((1, KD), jnp.float32),
        jax.ShapeDtypeStruct((1, K), jnp.int32),
    ],
)


def kernel(locF, Ck):
    X = locF[:, 0, :]
    xt = X.T
    x_flat = X.reshape(N * D)
    c = Ck[0]
    idx_lo = _assign_lo(c, xt).reshape(NH)
    s_lo, c_lo = _get_segsum(0)(x_flat, idx_lo)
    idx_hi = _assign_hi(c, xt).reshape(NH)
    s_hi, c_hi = _get_segsum(NH)(x_flat, idx_hi)
    ck1, n_items = _combine(s_lo, s_hi, c_lo, c_hi)
    return (ck1.reshape(1, K, D), n_items)
